# baseline (speedup 1.0000x reference)
import functools

import jax
import jax.numpy as jnp
from jax import lax
from jax.experimental import pallas as pl
from jax.experimental.pallas import tpu as pltpu
from jax.experimental.pallas import tpu_sc as plsc

D = 64
SUB = 128
N_SUB = 8
CHUNK = SUB * N_SUB


def kernel(input_ids, weight):
    B, L = input_ids.shape
    btot = B * L
    info = plsc.get_sparse_core_info()
    nw = info.num_cores * info.num_subcores
    assert btot % (nw * CHUNK) == 0
    b_per_w = btot // nw
    n_chunks = b_per_w // CHUNK

    ids2d = input_ids.reshape(btot // SUB, SUB).astype(jnp.int32)

    mesh = plsc.VectorSubcoreMesh(core_axis_name="c", subcore_axis_name="s")

    @functools.partial(
        pl.kernel,
        out_type=jax.ShapeDtypeStruct((btot, D), jnp.float32),
        mesh=mesh,
        scratch_types=[
            pltpu.VMEM((N_SUB, SUB), jnp.int32),
            pltpu.VMEM((CHUNK, D), jnp.float32),
            pltpu.SemaphoreType.DMA,
        ],
        compiler_params=pltpu.CompilerParams(use_tc_tiling_on_sc=False),
    )
    def gather_kernel(ids_hbm, table_hbm, out_hbm, idx_v, rows_v, sem):
        wid = lax.axis_index("s") * info.num_cores + lax.axis_index("c")
        row_base = wid * (b_per_w // SUB)

        def chunk_body(i, carry):
            row_off = row_base + i * N_SUB
            pltpu.sync_copy(ids_hbm.at[pl.ds(row_off, N_SUB)], idx_v)
            copies = []
            for j in range(N_SUB):
                copies.append(
                    pltpu.async_copy(
                        table_hbm.at[idx_v.at[j]],
                        rows_v.at[pl.ds(j * SUB, SUB)],
                        sem,
                    )
                )
            for c in copies:
                c.wait()
            pltpu.sync_copy(
                rows_v, out_hbm.at[pl.ds(row_off * SUB, CHUNK)]
            )
            return carry

        lax.fori_loop(0, n_chunks, chunk_body, 0)

    out = gather_kernel(ids2d, weight)
    return out.reshape(B, L, D)


# speedup vs baseline: 1.0138x; 1.0138x over previous
import functools

import jax
import jax.numpy as jnp
from jax import lax
from jax.experimental import pallas as pl
from jax.experimental.pallas import tpu as pltpu
from jax.experimental.pallas import tpu_sc as plsc

D = 64
SUB = 128
N_SUB = 4
CHUNK = SUB * N_SUB


def kernel(input_ids, weight):
    B, L = input_ids.shape
    btot = B * L
    info = plsc.get_sparse_core_info()
    nc = info.num_cores
    nw = nc * info.num_subcores
    assert btot % (nw * CHUNK) == 0
    b_per_w = btot // nw
    n_chunks = b_per_w // CHUNK
    assert n_chunks % 2 == 0 and n_chunks >= 4

    ids2d = input_ids.reshape(btot // SUB, SUB).astype(jnp.int32)
    ids2d = jnp.concatenate([ids2d, jnp.zeros((N_SUB, SUB), jnp.int32)], axis=0)

    mesh = plsc.VectorSubcoreMesh(core_axis_name="c", subcore_axis_name="s")

    @functools.partial(
        pl.kernel,
        out_type=jax.ShapeDtypeStruct((btot, D), jnp.float32),
        mesh=mesh,
        scratch_types=[
            pltpu.VMEM((N_SUB, SUB), jnp.int32),
            pltpu.VMEM((N_SUB, SUB), jnp.int32),
            pltpu.VMEM((CHUNK, D), jnp.float32),
            pltpu.VMEM((CHUNK, D), jnp.float32),
            pltpu.SemaphoreType.DMA,
            pltpu.SemaphoreType.DMA,
            pltpu.SemaphoreType.DMA,
            pltpu.SemaphoreType.DMA,
            pltpu.SemaphoreType.DMA,
            pltpu.SemaphoreType.DMA,
        ],
        compiler_params=pltpu.CompilerParams(use_tc_tiling_on_sc=False),
    )
    def gather_kernel(ids_hbm, table_hbm, out_hbm,
                      idx0, idx1, rows0, rows1,
                      sg0, sg1, so0, so1, si0, si1):
        wid = lax.axis_index("s") * nc + lax.axis_index("c")
        row_base = wid * (b_per_w // SUB)

        idx = (idx0, idx1)
        rows = (rows0, rows1)
        sg = (sg0, sg1)
        so = (so0, so1)
        si = (si0, si1)

        def idx_copy(i, b):
            row_off = row_base + i * N_SUB
            return pltpu.make_async_copy(
                ids_hbm.at[pl.ds(row_off, N_SUB)], idx[b], si[b])

        def gather_copies(b):
            return [
                pltpu.make_async_copy(
                    table_hbm.at[idx[b].at[j]],
                    rows[b].at[pl.ds(j * SUB, SUB)],
                    sg[b])
                for j in range(N_SUB)
            ]

        def out_copy(i, b):
            row_off = row_base + i * N_SUB
            return pltpu.make_async_copy(
                rows[b], out_hbm.at[pl.ds(row_off * SUB, CHUNK)], so[b])

        pltpu.sync_copy(ids_hbm.at[pl.ds(row_base, N_SUB)], idx0)
        for c in gather_copies(0):
            c.start()
        pltpu.sync_copy(ids_hbm.at[pl.ds(row_base + N_SUB, N_SUB)], idx1)
        for c in gather_copies(0):
            c.wait()
        out_copy(0, 0).start()
        idx_copy(2, 0).start()
        for c in gather_copies(1):
            c.start()

        def pair(g, carry):
            i0 = 2 * g
            for i, b in ((i0, 0), (i0 + 1, 1)):
                ob = 1 - b
                for c in gather_copies(ob):
                    c.wait()
                out_copy(i - 1, ob).start()
                idx_copy(i + 1, ob).start()
                out_copy(i - 2, b).wait()
                idx_copy(i, b).wait()
                for c in gather_copies(b):
                    c.start()
            return carry

        lax.fori_loop(1, n_chunks // 2, pair, 0)

        last = n_chunks - 1
        for c in gather_copies(1):
            c.wait()
        out_copy(last, 1).start()
        out_copy(last - 1, 0).wait()
        idx_copy(n_chunks, 0).wait()
        out_copy(last, 1).wait()

    out = gather_kernel(ids2d, weight)
    return out.reshape(B, L, D)
